# Initial kernel scaffold; baseline (speedup 1.0000x reference)
import jax
import jax.numpy as jnp
from jax.experimental import pallas as pl


def kernel(target, pred):
    raise NotImplementedError("write your pallas kernel here")


# speedup vs baseline: 1.2088x; 1.2088x over previous
import jax
import jax.numpy as jnp
from jax.experimental import pallas as pl

_SMOOTHING = 0.1
_BC = 1024


def kernel(target, pred):
    b, k = pred.shape
    low = _SMOOTHING / k
    hi = low + (1.0 - _SMOOTHING)

    def body(t_ref, o_ref):
        j = pl.program_id(0)
        cols = j * _BC + jax.lax.broadcasted_iota(jnp.int32, (b, _BC), 1)
        mask = cols == t_ref[:, :]
        o_ref[:, :] = jnp.where(mask, hi, low).astype(o_ref.dtype)

    t2 = target.reshape(b, 1)
    return pl.pallas_call(
        body,
        grid=(pl.cdiv(k, _BC),),
        in_specs=[pl.BlockSpec((b, 1), lambda j: (0, 0))],
        out_specs=pl.BlockSpec((b, _BC), lambda j: (0, j)),
        out_shape=jax.ShapeDtypeStruct((b, k), pred.dtype),
    )(t2)
